# Initial kernel scaffold; baseline (speedup 1.0000x reference)
import jax
import jax.numpy as jnp
from jax.experimental import pallas as pl


def kernel(A2O_aspect_hidden_states, A2O_opinion_hidden_states, O2A_aspect_hidden_states, O2A_opinion_hidden_states, W_A2O, b_A2O, W_O2A, b_O2A, asp_idx_a2o, opi_idx_a2o, asp_idx_o2a, opi_idx_o2a, sentiment_labels):
    raise NotImplementedError("write your pallas kernel here")


# speedup vs baseline: 58.1473x; 58.1473x over previous
import functools

import jax
import jax.numpy as jnp
from jax import lax
from jax.experimental import pallas as pl
from jax.experimental.pallas import tpu as pltpu
from jax.experimental.pallas import tpu_sc as plsc

_B, _N, _H, _K = 4, 2048, 768, 256
_BIG = 2 ** 30


def _sc_gather(aa2, ao2, oo2, oa2, idx_all):
    info = plsc.get_sparse_core_info()
    nc = info.num_cores
    mesh = plsc.VectorSubcoreMesh(core_axis_name="c", subcore_axis_name="s")

    @functools.partial(
        pl.kernel,
        mesh=mesh,
        out_type=jax.ShapeDtypeStruct((_B * 6 * _K, _H), jnp.float32),
        scratch_types=[
            pltpu.VMEM((64,), jnp.int32),
            pltpu.VMEM((64, _H), jnp.float32),
            pltpu.VMEM((32,), jnp.int32),
            pltpu.VMEM((32, _H), jnp.float32),
            pltpu.SemaphoreType.DMA,
        ],
    )
    def gk(aa_h, ao_h, oo_h, oa_h, idx_h, out_h, idx64, rows64, idx32, rows32, sem):
        w = lax.axis_index("s") * nc + lax.axis_index("c")
        b = w // 8
        r = w % 8
        slot = r // 4
        chunk4 = r % 4
        chunk8 = r
        pltpu.sync_copy(idx_h.at[w], idx64)
        pltpu.async_copy(aa_h.at[idx64], rows64, sem).wait()
        pltpu.sync_copy(rows64, out_h.at[pl.ds((b * 6 + slot) * 256 + chunk4 * 64, 64)])
        pltpu.sync_copy(idx_h.at[32 + w], idx64)
        pltpu.async_copy(oo_h.at[idx64], rows64, sem).wait()
        pltpu.sync_copy(rows64, out_h.at[pl.ds((b * 6 + 3 + slot) * 256 + chunk4 * 64, 64)])
        pltpu.sync_copy(idx_h.at[64 + w, pl.ds(0, 32)], idx32)
        pltpu.async_copy(ao_h.at[idx32], rows32, sem).wait()
        pltpu.sync_copy(rows32, out_h.at[pl.ds((b * 6 + 2) * 256 + chunk8 * 32, 32)])
        pltpu.sync_copy(idx_h.at[96 + w, pl.ds(0, 32)], idx32)
        pltpu.async_copy(oa_h.at[idx32], rows32, sem).wait()
        pltpu.sync_copy(rows32, out_h.at[pl.ds((b * 6 + 5) * 256 + chunk8 * 32, 32)])

    return gk(aa2, ao2, oo2, oa2, idx_all)


def _tc_body(g_ref, ia_ref, ja_ref, io_ref, jo_ref, lab_ref, wa_ref, wo_ref,
             bp_ref, ao0_ref, oo0_ref, fl_ref, pred_ref, loss_ref):
    b = pl.program_id(0)
    K, H, N = _K, _H, _N
    earlier = lax.broadcasted_iota(jnp.int32, (K, K), 0) < \
        lax.broadcasted_iota(jnp.int32, (K, K), 1)
    iota_nk = lax.broadcasted_iota(jnp.int32, (N, K), 0)

    def direction(Ha, Ho, Oo, Aa, asp_row, opi_row, w_ref, b_row, o0):
        w_top = w_ref[:H, :]
        w_bot = w_ref[H:, :]
        asp_col = asp_row.reshape(K, 1)
        opi_col = opi_row.reshape(K, 1)
        s = lax.dot_general(Ha, Ho, (((1,), (1,)), ((), ())),
                            preferred_element_type=jnp.float32) * 0.01
        neq = (asp_col != opi_row).astype(jnp.float32)
        ssc = s * neq
        dup_o = jnp.any((opi_col == opi_row) & earlier, axis=0, keepdims=True)
        colmask = jnp.where(dup_o, 0.0, 1.0)
        row_sum = jnp.sum(ssc * colmask, axis=1, keepdims=True)
        a = jnp.exp(ssc - row_sum) * neq
        max_a = jnp.max(a, axis=1, keepdims=True)
        has = max_a > 0.0
        is_max = (a == max_a) & (neq > 0.0) & has
        opi_b = jnp.broadcast_to(opi_row, (K, K))
        jstar = jnp.min(jnp.where(is_max, opi_b, _BIG), axis=1, keepdims=True)
        sel = (is_max & (opi_b == jstar)).astype(jnp.float32)
        cnt = jnp.sum(sel, axis=1, keepdims=True)
        proj_o = jnp.dot(Oo, w_bot, preferred_element_type=jnp.float32)
        psel = jnp.dot(sel, proj_o, preferred_element_type=jnp.float32) / \
            jnp.maximum(cnt, 1.0)
        proj0 = jnp.dot(o0, w_bot, preferred_element_type=jnp.float32)
        psel = jnp.where(has, psel, jnp.broadcast_to(proj0, (K, 3)))
        g_a = jnp.dot(Aa, w_top, preferred_element_type=jnp.float32)
        l = g_a + psel + b_row
        dup_a = jnp.any((asp_col == asp_row) & earlier, axis=0, keepdims=True)
        rowmask = jnp.where(dup_a, 0.0, 1.0)
        oh_t = (iota_nk == jnp.broadcast_to(asp_row, (N, K))).astype(jnp.float32) * rowmask
        return jnp.dot(oh_t, l, preferred_element_type=jnp.float32)

    fl1 = direction(g_ref[0, 0], g_ref[0, 1], g_ref[0, 2], g_ref[0, 0],
                    ia_ref[0], ja_ref[0], wa_ref, bp_ref[0:1, 0:3], ao0_ref[0])
    fl2 = direction(g_ref[0, 3], g_ref[0, 4], g_ref[0, 4], g_ref[0, 5],
                    io_ref[0], jo_ref[0], wo_ref, bp_ref[1:2, 0:3], oo0_ref[0])
    fl = 0.5 * (fl1 + fl2)
    valid = jnp.sum(jnp.abs(fl), axis=1, keepdims=True) > 0.0
    m = jnp.max(fl, axis=1, keepdims=True)
    ex = jnp.exp(fl - m)
    logp = fl - m - jnp.log(jnp.sum(ex, axis=1, keepdims=True))
    lab_col = lab_ref[0].reshape(N, 1)
    nll = -jnp.where(lab_col == 0, logp[:, 0:1],
                     jnp.where(lab_col == 1, logp[:, 1:2], logp[:, 2:3]))
    wlab = jnp.where(lab_col == 0, 1.0, jnp.where(lab_col == 1, 2.0, 4.0))
    wl = wlab * valid.astype(jnp.float32)
    loss_b = (jnp.sum(nll * wl) / jnp.maximum(jnp.sum(wl), 1e-6)).reshape(1, 1)
    f0, f1, f2 = fl[:, 0:1], fl[:, 1:2], fl[:, 2:3]
    p01 = jnp.where(f1 > f0, 1, 0)
    pidx = jnp.where(f2 > jnp.maximum(f0, f1), 2, p01)
    pred = jnp.where(valid, pidx, -1)
    fl_ref[0] = fl
    pred_ref[0] = pred.reshape(1, N)

    @pl.when(b == 0)
    def _():
        loss_ref[:, :] = jnp.zeros((1, 1), jnp.float32)

    loss_ref[:, :] += loss_b


_TC_GRID = (_B,)
_TC_IN_SPECS = [
    pl.BlockSpec((1, 6, _K, _H), lambda b: (b, 0, 0, 0)),
    pl.BlockSpec((1, 1, _K), lambda b: (b, 0, 0)),
    pl.BlockSpec((1, 1, _K), lambda b: (b, 0, 0)),
    pl.BlockSpec((1, 1, _K), lambda b: (b, 0, 0)),
    pl.BlockSpec((1, 1, _K), lambda b: (b, 0, 0)),
    pl.BlockSpec((1, 1, _N), lambda b: (b, 0, 0)),
    pl.BlockSpec((2 * _H, 3), lambda b: (0, 0)),
    pl.BlockSpec((2 * _H, 3), lambda b: (0, 0)),
    pl.BlockSpec((8, 128), lambda b: (0, 0)),
    pl.BlockSpec((1, 1, _H), lambda b: (b, 0, 0)),
    pl.BlockSpec((1, 1, _H), lambda b: (b, 0, 0)),
]
_TC_OUT_SPECS = [
    pl.BlockSpec((1, _N, 3), lambda b: (b, 0, 0)),
    pl.BlockSpec((1, 1, _N), lambda b: (b, 0, 0)),
    pl.BlockSpec((1, 1), lambda b: (0, 0)),
]
_TC_OUT_SHAPE = [
    jax.ShapeDtypeStruct((_B, _N, 3), jnp.float32),
    jax.ShapeDtypeStruct((_B, 1, _N), jnp.int32),
    jax.ShapeDtypeStruct((1, 1), jnp.float32),
]


def _pack_indices(ia, ja, io, jo):
    offs = (jnp.arange(_B, dtype=jnp.int32) * _N)[:, None]
    fia, fja = ia + offs, ja + offs
    fio, fjo = io + offs, jo + offs
    p0 = jnp.stack([fia, fja], axis=1).reshape(32, 64)
    p1 = jnp.stack([fio, fjo], axis=1).reshape(32, 64)
    p2 = jnp.pad(fja.reshape(32, 32), ((0, 0), (0, 32)))
    p3 = jnp.pad(fio.reshape(32, 32), ((0, 0), (0, 32)))
    return jnp.concatenate([p0, p1, p2, p3], axis=0)


def kernel(A2O_aspect_hidden_states, A2O_opinion_hidden_states,
           O2A_aspect_hidden_states, O2A_opinion_hidden_states,
           W_A2O, b_A2O, W_O2A, b_O2A,
           asp_idx_a2o, opi_idx_a2o, asp_idx_o2a, opi_idx_o2a,
           sentiment_labels):
    aa, ao = A2O_aspect_hidden_states, A2O_opinion_hidden_states
    oa, oo = O2A_aspect_hidden_states, O2A_opinion_hidden_states
    ia = asp_idx_a2o.astype(jnp.int32)
    ja = opi_idx_a2o.astype(jnp.int32)
    io = asp_idx_o2a.astype(jnp.int32)
    jo = opi_idx_o2a.astype(jnp.int32)
    idx_all = _pack_indices(ia, ja, io, jo)
    g = _sc_gather(aa.reshape(_B * _N, _H), ao.reshape(_B * _N, _H),
                   oo.reshape(_B * _N, _H), oa.reshape(_B * _N, _H),
                   idx_all).reshape(_B, 6, _K, _H)
    bpad = jnp.zeros((8, 128), jnp.float32).at[0, :3].set(b_A2O).at[1, :3].set(b_O2A)
    fl, pred, loss = pl.pallas_call(
        _tc_body,
        grid=_TC_GRID,
        in_specs=_TC_IN_SPECS,
        out_specs=_TC_OUT_SPECS,
        out_shape=_TC_OUT_SHAPE,
    )(g, ia.reshape(_B, 1, _K), ja.reshape(_B, 1, _K),
      io.reshape(_B, 1, _K), jo.reshape(_B, 1, _K),
      sentiment_labels.astype(jnp.int32).reshape(_B, 1, _N),
      W_A2O, W_O2A, bpad, ao[:, 0:1, :], oo[:, 0:1, :])
    return fl, pred.reshape(_B, _N), loss[0, 0]


# speedup vs baseline: 101.3882x; 1.7436x over previous
import functools

import jax
import jax.numpy as jnp
from jax import lax
from jax.experimental import pallas as pl
from jax.experimental.pallas import tpu as pltpu
from jax.experimental.pallas import tpu_sc as plsc

_B, _N, _H, _K = 4, 2048, 768, 256
_BIG = 2 ** 30


def _sc_gather(aa2, ao2, oo2, oa2, ija):
    info = plsc.get_sparse_core_info()
    nc = info.num_cores
    mesh = plsc.VectorSubcoreMesh(core_axis_name="c", subcore_axis_name="s")

    @functools.partial(
        pl.kernel,
        mesh=mesh,
        out_type=jax.ShapeDtypeStruct((_B * 6 * _K, _H), jnp.float32),
        scratch_types=[
            pltpu.VMEM((64,), jnp.int32),
            pltpu.VMEM((64, _H), jnp.float32),
            pltpu.VMEM((32,), jnp.int32),
            pltpu.VMEM((32, _H), jnp.float32),
            pltpu.SemaphoreType.DMA,
        ],
    )
    def gk(aa_h, ao_h, oo_h, oa_h, ija_h, out_h, idx64, rows64, idx32, rows32, sem):
        w = lax.axis_index("s") * nc + lax.axis_index("c")
        b = w // 8
        r = w % 8
        slot = r // 4
        chunk4 = r % 4
        chunk8 = r
        base = b * _N

        def bump(ref, n):
            for i in range(n // 16):
                ref[pl.ds(i * 16, 16)] = ref[pl.ds(i * 16, 16)] + base

        pltpu.sync_copy(ija_h.at[slot, b, pl.ds(chunk4 * 64, 64)], idx64)
        bump(idx64, 64)
        pltpu.async_copy(aa_h.at[idx64], rows64, sem).wait()
        pltpu.sync_copy(rows64, out_h.at[pl.ds((b * 6 + slot) * 256 + chunk4 * 64, 64)])
        pltpu.sync_copy(ija_h.at[2 + slot, b, pl.ds(chunk4 * 64, 64)], idx64)
        bump(idx64, 64)
        pltpu.async_copy(oo_h.at[idx64], rows64, sem).wait()
        pltpu.sync_copy(rows64, out_h.at[pl.ds((b * 6 + 3 + slot) * 256 + chunk4 * 64, 64)])
        pltpu.sync_copy(ija_h.at[1, b, pl.ds(chunk8 * 32, 32)], idx32)
        bump(idx32, 32)
        pltpu.async_copy(ao_h.at[idx32], rows32, sem).wait()
        pltpu.sync_copy(rows32, out_h.at[pl.ds((b * 6 + 2) * 256 + chunk8 * 32, 32)])
        pltpu.sync_copy(ija_h.at[2, b, pl.ds(chunk8 * 32, 32)], idx32)
        bump(idx32, 32)
        pltpu.async_copy(oa_h.at[idx32], rows32, sem).wait()
        pltpu.sync_copy(rows32, out_h.at[pl.ds((b * 6 + 5) * 256 + chunk8 * 32, 32)])

    return gk(aa2, ao2, oo2, oa2, ija)


def _tc_body(g_ref, ia_ref, ja_ref, io_ref, jo_ref, lab_ref, wat_ref, wot_ref,
             bp_ref, ao0_ref, oo0_ref, fl_ref, pred_ref, loss_ref):
    b = pl.program_id(0)
    K, H, N = _K, _H, _N
    lower = lax.broadcasted_iota(jnp.int32, (K, K), 1) < \
        lax.broadcasted_iota(jnp.int32, (K, K), 0)
    iota_kn = lax.broadcasted_iota(jnp.int32, (K, N), 1)
    nt = (((1,), (1,)), ((), ()))

    def direction(Ha, Ho, Oo, Aa, asp_row, opi_row, wt_ref, b_col, o0):
        w_top_t = wt_ref[:, :H]
        w_bot_t = wt_ref[:, H:]
        asp_col = asp_row.reshape(K, 1)
        opi_col = opi_row.reshape(K, 1)
        st = lax.dot_general(Ho, Ha, nt, preferred_element_type=jnp.float32) * 0.01
        neqt = (opi_col != asp_row).astype(jnp.float32)
        ssct = st * neqt
        dup_o = jnp.any((opi_col == opi_row) & lower, axis=1, keepdims=True)
        colmask = jnp.where(dup_o, 0.0, 1.0)
        row_sum = jnp.sum(ssct * colmask, axis=0, keepdims=True)
        at = jnp.exp(ssct - row_sum) * neqt
        max_a = jnp.max(at, axis=0, keepdims=True)
        has = max_a > 0.0
        is_max = (at == max_a) & (neqt > 0.0) & has
        opi_b = jnp.broadcast_to(opi_col, (K, K))
        jstar = jnp.min(jnp.where(is_max, opi_b, _BIG), axis=0, keepdims=True)
        sel = (is_max & (opi_b == jstar)).astype(jnp.float32)
        cnt = jnp.sum(sel, axis=0, keepdims=True)
        proj_ot = lax.dot_general(w_bot_t, Oo, nt,
                                  preferred_element_type=jnp.float32)
        pselt = jnp.dot(proj_ot, sel, preferred_element_type=jnp.float32) / \
            jnp.maximum(cnt, 1.0)
        proj0t = lax.dot_general(w_bot_t, o0, nt,
                                 preferred_element_type=jnp.float32)
        pselt = jnp.where(has, pselt, jnp.broadcast_to(proj0t, (3, K)))
        g_at = lax.dot_general(w_top_t, Aa, nt,
                               preferred_element_type=jnp.float32)
        lt = g_at + pselt + b_col
        dup_a = jnp.any((asp_col == asp_row) & lower, axis=1, keepdims=True)
        rowmask = jnp.where(dup_a, 0.0, 1.0)
        oh = (asp_col == iota_kn).astype(jnp.float32) * rowmask
        return jnp.dot(lt, oh, preferred_element_type=jnp.float32)

    fl1 = direction(g_ref[0, 0], g_ref[0, 1], g_ref[0, 2], g_ref[0, 0],
                    ia_ref[0], ja_ref[0], wat_ref, bp_ref[0:3, 0:1], ao0_ref[0])
    fl2 = direction(g_ref[0, 3], g_ref[0, 4], g_ref[0, 4], g_ref[0, 5],
                    io_ref[0], jo_ref[0], wot_ref, bp_ref[0:3, 1:2], oo0_ref[0])
    fl = 0.5 * (fl1 + fl2)
    valid = jnp.sum(jnp.abs(fl), axis=0, keepdims=True) > 0.0
    m = jnp.max(fl, axis=0, keepdims=True)
    ex = jnp.exp(fl - m)
    logp = fl - m - jnp.log(jnp.sum(ex, axis=0, keepdims=True))
    lab_row = lab_ref[0]
    nll = -jnp.where(lab_row == 0, logp[0:1, :],
                     jnp.where(lab_row == 1, logp[1:2, :], logp[2:3, :]))
    wlab = jnp.where(lab_row == 0, 1.0, jnp.where(lab_row == 1, 2.0, 4.0))
    wl = wlab * valid.astype(jnp.float32)
    loss_b = (jnp.sum(nll * wl) / jnp.maximum(jnp.sum(wl), 1e-6)).reshape(1, 1)
    f0, f1, f2 = fl[0:1, :], fl[1:2, :], fl[2:3, :]
    p01 = jnp.where(f1 > f0, 1, 0)
    pidx = jnp.where(f2 > jnp.maximum(f0, f1), 2, p01)
    pred = jnp.where(valid, pidx, -1)
    fl_ref[0] = fl
    pred_ref[0] = pred

    @pl.when(b == 0)
    def _():
        loss_ref[:, :] = jnp.zeros((1, 1), jnp.float32)

    loss_ref[:, :] += loss_b


_TC_GRID = (_B,)
_TC_IN_SPECS = [
    pl.BlockSpec((1, 6, _K, _H), lambda b: (b, 0, 0, 0)),
    pl.BlockSpec((1, 1, _K), lambda b: (b, 0, 0)),
    pl.BlockSpec((1, 1, _K), lambda b: (b, 0, 0)),
    pl.BlockSpec((1, 1, _K), lambda b: (b, 0, 0)),
    pl.BlockSpec((1, 1, _K), lambda b: (b, 0, 0)),
    pl.BlockSpec((1, 1, _N), lambda b: (b, 0, 0)),
    pl.BlockSpec((3, 2 * _H), lambda b: (0, 0)),
    pl.BlockSpec((3, 2 * _H), lambda b: (0, 0)),
    pl.BlockSpec((8, 128), lambda b: (0, 0)),
    pl.BlockSpec((1, 1, _H), lambda b: (b, 0, 0)),
    pl.BlockSpec((1, 1, _H), lambda b: (b, 0, 0)),
]
_TC_OUT_SPECS = [
    pl.BlockSpec((1, 3, _N), lambda b: (b, 0, 0)),
    pl.BlockSpec((1, 1, _N), lambda b: (b, 0, 0)),
    pl.BlockSpec((1, 1), lambda b: (0, 0)),
]
_TC_OUT_SHAPE = [
    jax.ShapeDtypeStruct((_B, 3, _N), jnp.float32),
    jax.ShapeDtypeStruct((_B, 1, _N), jnp.int32),
    jax.ShapeDtypeStruct((1, 1), jnp.float32),
]


def kernel(A2O_aspect_hidden_states, A2O_opinion_hidden_states,
           O2A_aspect_hidden_states, O2A_opinion_hidden_states,
           W_A2O, b_A2O, W_O2A, b_O2A,
           asp_idx_a2o, opi_idx_a2o, asp_idx_o2a, opi_idx_o2a,
           sentiment_labels):
    aa, ao = A2O_aspect_hidden_states, A2O_opinion_hidden_states
    oa, oo = O2A_aspect_hidden_states, O2A_opinion_hidden_states
    ija = jnp.stack([asp_idx_a2o, opi_idx_a2o,
                     asp_idx_o2a, opi_idx_o2a]).astype(jnp.int32)
    g = _sc_gather(aa.reshape(_B * _N, _H), ao.reshape(_B * _N, _H),
                   oo.reshape(_B * _N, _H), oa.reshape(_B * _N, _H),
                   ija).reshape(_B, 6, _K, _H)
    bpad = jnp.zeros((8, 128), jnp.float32).at[:3, 0].set(b_A2O).at[:3, 1].set(b_O2A)
    fl_t, pred, loss = pl.pallas_call(
        _tc_body,
        grid=_TC_GRID,
        in_specs=_TC_IN_SPECS,
        out_specs=_TC_OUT_SPECS,
        out_shape=_TC_OUT_SHAPE,
    )(g, ija[0].reshape(_B, 1, _K), ija[1].reshape(_B, 1, _K),
      ija[2].reshape(_B, 1, _K), ija[3].reshape(_B, 1, _K),
      sentiment_labels.astype(jnp.int32).reshape(_B, 1, _N),
      W_A2O.T, W_O2A.T, bpad, ao[:, 0:1, :], oo[:, 0:1, :])
    return jnp.swapaxes(fl_t, 1, 2), pred.reshape(_B, _N), loss[0, 0]
